# Initial kernel scaffold; baseline (speedup 1.0000x reference)
import jax
import jax.numpy as jnp
from jax.experimental import pallas as pl


def kernel(x, tok_table):
    raise NotImplementedError("write your pallas kernel here")


# speedup vs baseline: 2.1332x; 2.1332x over previous
import functools
import math

import jax
import jax.numpy as jnp
from jax import lax
from jax.experimental import pallas as pl
from jax.experimental.pallas import tpu as pltpu
from jax.experimental.pallas import tpu_sc as plsc

VOCAB = 100000
D = 1024
S = 8192
B = 4

NC = 2
NS = 16
NW = NC * NS
S_PER_W = S // NW
C = 32
NCH = S_PER_W // C

_LN10000 = math.log(10000.0)


_POS_BS = 1024


def _pos_body(out_ref):
    i = pl.program_id(0)
    row = (
        lax.broadcasted_iota(jnp.int32, (_POS_BS, D), 0) + i * _POS_BS
    ).astype(jnp.float32)
    col = lax.broadcasted_iota(jnp.int32, (_POS_BS, D), 1)
    i_val = ((col >> 1) << 1).astype(jnp.float32)
    denom = jnp.exp(i_val * (_LN10000 / float(D)))
    angle = row / denom
    out_ref[...] = jnp.where((col & 1) == 0, jnp.sin(angle), jnp.cos(angle))


def _make_pos():
    return pl.pallas_call(
        _pos_body,
        out_shape=jax.ShapeDtypeStruct((S, D), jnp.float32),
        grid=(S // _POS_BS,),
        out_specs=pl.BlockSpec((_POS_BS, D), lambda i: (i, 0)),
    )()


def _sc_body(pos_hbm, x_hbm, tab_hbm, out_hbm, idx_v, pos_v, rows_v, sem):
    cid = lax.axis_index("c")
    sid = lax.axis_index("s")
    wid = sid * NC + cid
    w0 = wid * S_PER_W

    def chunk_body(ch, _):
        s0 = pl.multiple_of(w0 + ch * C, C)
        pltpu.sync_copy(pos_hbm.at[pl.ds(s0, C), :], pos_v)
        for b in range(B):
            base = pl.multiple_of(b * S + s0, C)
            pltpu.sync_copy(x_hbm.at[pl.ds(base, C)], idx_v)
            pltpu.async_copy(tab_hbm.at[idx_v], rows_v, sem).wait()

            def add_row(r, _):
                def add_slice(j, _):
                    off = j * 16
                    rows_v[r, pl.ds(off, 16)] = (
                        rows_v[r, pl.ds(off, 16)] + pos_v[r, pl.ds(off, 16)]
                    )
                    return 0

                lax.fori_loop(0, D // 16, add_slice, 0, unroll=4)
                return 0

            lax.fori_loop(0, C, add_row, 0)
            pltpu.sync_copy(rows_v, out_hbm.at[pl.ds(base, C), :])
        return 0

    lax.fori_loop(0, NCH, chunk_body, 0)


def _make_sc():
    mesh = plsc.VectorSubcoreMesh(core_axis_name="c", subcore_axis_name="s")
    return functools.partial(
        pl.kernel,
        mesh=mesh,
        out_type=jax.ShapeDtypeStruct((B * S, D), jnp.float32),
        scratch_types=[
            pltpu.VMEM((C,), jnp.int32),
            pltpu.VMEM((C, D), jnp.float32),
            pltpu.VMEM((C, D), jnp.float32),
            pltpu.SemaphoreType.DMA,
        ],
    )(_sc_body)


@jax.jit
def kernel(x, tok_table):
    pos = _make_pos()
    out = _make_sc()(pos, x.reshape(B * S), tok_table)
    return out.reshape(B, S, D)


# speedup vs baseline: 2.9337x; 1.3753x over previous
import functools
import math

import jax
import jax.numpy as jnp
from jax import lax
from jax.experimental import pallas as pl
from jax.experimental.pallas import tpu as pltpu
from jax.experimental.pallas import tpu_sc as plsc

VOCAB = 100000
D = 1024
S = 8192
B = 4

NC = 2
NS = 16
NW = NC * NS
S_PER_W = S // NW
C = 32
NCH = S_PER_W // C

NH = S // 64
NL = 64

_LN10000 = math.log(10000.0)


def _inv_denom_cols(shape, dim):
    col = lax.broadcasted_iota(jnp.int32, shape, dim)
    i_val = ((col >> 1) << 1).astype(jnp.float32)
    return jnp.exp(i_val * (_LN10000 / float(D)))


def _tab_body(sinh_ref, cosh_ref, sinl_ref, cosl_ref):
    ah = (
        lax.broadcasted_iota(jnp.int32, (NH, 1, D), 0) * 64
    ).astype(jnp.float32) / _inv_denom_cols((NH, 1, D), 2)
    sinh_ref[...] = jnp.sin(ah)
    cosh_ref[...] = jnp.cos(ah)
    al = lax.broadcasted_iota(jnp.int32, (NL, D), 0).astype(
        jnp.float32
    ) / _inv_denom_cols((NL, D), 1)
    sinl_ref[...] = jnp.sin(al)
    cosl_ref[...] = jnp.cos(al)


def _combine_body(sinh_ref, cosh_ref, sinl_ref, cosl_ref, out_ref):
    sh = sinh_ref[0]
    ch_ = cosh_ref[0]
    sl = sinl_ref[...]
    cl = cosl_ref[...]
    col = lax.broadcasted_iota(jnp.int32, (NL, D), 1)
    even = (col & 1) == 0
    out_ref[...] = jnp.where(even, sh * cl + ch_ * sl, ch_ * cl - sh * sl)


def _make_pos():
    f32 = jnp.float32
    tabs = pl.pallas_call(
        _tab_body,
        out_shape=[
            jax.ShapeDtypeStruct((NH, 1, D), f32),
            jax.ShapeDtypeStruct((NH, 1, D), f32),
            jax.ShapeDtypeStruct((NL, D), f32),
            jax.ShapeDtypeStruct((NL, D), f32),
        ],
    )()
    row_spec = pl.BlockSpec((1, 1, D), lambda h: (h, 0, 0))
    full_spec = pl.BlockSpec((NL, D), lambda h: (0, 0))
    return pl.pallas_call(
        _combine_body,
        out_shape=jax.ShapeDtypeStruct((S, D), f32),
        grid=(NH,),
        in_specs=[row_spec, row_spec, full_spec, full_spec],
        out_specs=pl.BlockSpec((NL, D), lambda h: (h, 0)),
    )(*tabs)


def _add_pos(rows_v, pos_v):
    def add_row(r, _):
        def add_slice(j, _):
            off = j * 16
            rows_v[r, pl.ds(off, 16)] = (
                rows_v[r, pl.ds(off, 16)] + pos_v[r, pl.ds(off, 16)]
            )
            return 0

        lax.fori_loop(0, D // 16, add_slice, 0, unroll=4)
        return 0

    lax.fori_loop(0, C, add_row, 0)


def _sc_body(
    pos_hbm, x_hbm, tab_hbm, out_hbm, idx_v, pos_v, r0, r1, g0, g1, sem
):
    cid = lax.axis_index("c")
    sid = lax.axis_index("s")
    wid = sid * NC + cid
    w0 = wid * S_PER_W

    rows = (r0, r1)
    gsem = (g0, g1)

    def gather(ch, b, buf):
        idx = idx_v.at[b, pl.ds(pl.multiple_of(ch * C, C), C)]
        pltpu.make_async_copy(tab_hbm.at[idx], rows[buf], gsem[buf]).start()

    def gather_wait(b, buf):
        idx = idx_v.at[b, pl.ds(0, C)]
        pltpu.make_async_copy(tab_hbm.at[idx], rows[buf], gsem[buf]).wait()

    for b in range(B):
        pltpu.sync_copy(
            x_hbm.at[pl.ds(pl.multiple_of(b * S + w0, C), S_PER_W)],
            idx_v.at[b],
        )
    pltpu.sync_copy(pos_hbm.at[pl.ds(pl.multiple_of(w0, C), C), :], pos_v)
    gather(0, 0, 0)

    def chunk_body(i, _):
        s0 = pl.multiple_of(w0 + i * C, C)
        for k in range(B):
            buf = k % 2
            nbuf = 1 - buf
            if k < B - 1:
                gather(i, k + 1, nbuf)
            else:

                @pl.when(i < NCH - 1)
                def _():
                    gather(i + 1, 0, nbuf)

            if k == 0:

                @pl.when(i > 0)
                def _():
                    pltpu.sync_copy(pos_hbm.at[pl.ds(s0, C), :], pos_v)

            gather_wait(k, buf)
            _add_pos(rows[buf], pos_v)
            base = pl.multiple_of(k * S + s0, C)
            pltpu.sync_copy(rows[buf], out_hbm.at[pl.ds(base, C), :])
        return 0

    lax.fori_loop(0, NCH, chunk_body, 0)


def _make_sc():
    mesh = plsc.VectorSubcoreMesh(core_axis_name="c", subcore_axis_name="s")
    return functools.partial(
        pl.kernel,
        mesh=mesh,
        out_type=jax.ShapeDtypeStruct((B * S, D), jnp.float32),
        scratch_types=[
            pltpu.VMEM((B, S_PER_W), jnp.int32),
            pltpu.VMEM((C, D), jnp.float32),
            pltpu.VMEM((C, D), jnp.float32),
            pltpu.VMEM((C, D), jnp.float32),
            pltpu.SemaphoreType.DMA,
            pltpu.SemaphoreType.DMA,
            pltpu.SemaphoreType.DMA,
        ],
    )(_sc_body)


@jax.jit
def kernel(x, tok_table):
    pos = _make_pos()
    out = _make_sc()(pos, x.reshape(B * S), tok_table)
    return out.reshape(B, S, D)


# speedup vs baseline: 3.9637x; 1.3511x over previous
import functools
import math

import jax
import jax.numpy as jnp
from jax import lax
from jax.experimental import pallas as pl
from jax.experimental.pallas import tpu as pltpu
from jax.experimental.pallas import tpu_sc as plsc

VOCAB = 100000
D = 1024
S = 8192
B = 4

NC = 2
NS = 16
NW = NC * NS
S_PER_W = S // NW
C = 32
NCH = S_PER_W // C

NH = S // 64
NL = 64

_LN10000 = math.log(10000.0)


_HPB = 8
_PBS = _HPB * NL


def _inv_denom_cols(shape, dim):
    col = lax.broadcasted_iota(jnp.int32, shape, dim)
    i_val = ((col >> 1) << 1).astype(jnp.float32)
    return jnp.exp(i_val * (_LN10000 / float(D)))


def _pos_body(out_ref, sinl_ref, cosl_ref, sinh_ref, cosh_ref):
    g = pl.program_id(0)

    @pl.when(g == 0)
    def _():
        al = lax.broadcasted_iota(jnp.int32, (NL, D), 0).astype(
            jnp.float32
        ) / _inv_denom_cols((NL, D), 1)
        sinl_ref[...] = jnp.sin(al)
        cosl_ref[...] = jnp.cos(al)
        ah = (
            lax.broadcasted_iota(jnp.int32, (NH, D), 0) * 64
        ).astype(jnp.float32) / _inv_denom_cols((NH, D), 1)
        sinh_ref[...] = jnp.sin(ah)
        cosh_ref[...] = jnp.cos(ah)

    hs = pl.ds(g * _HPB, _HPB)
    sh = sinh_ref[hs, :][:, None, :]
    ch_ = cosh_ref[hs, :][:, None, :]
    sl = sinl_ref[...][None, :, :]
    cl = cosl_ref[...][None, :, :]
    col = lax.broadcasted_iota(jnp.int32, (_HPB, NL, D), 2)
    even = (col & 1) == 0
    res = jnp.where(even, sh * cl + ch_ * sl, ch_ * cl - sh * sl)
    out_ref[...] = res.reshape(_PBS, D)


def _make_pos():
    return pl.pallas_call(
        _pos_body,
        out_shape=jax.ShapeDtypeStruct((S, D), jnp.float32),
        grid=(S // _PBS,),
        out_specs=pl.BlockSpec((_PBS, D), lambda g: (g, 0)),
        scratch_shapes=[
            pltpu.VMEM((NL, D), jnp.float32),
            pltpu.VMEM((NL, D), jnp.float32),
            pltpu.VMEM((NH, D), jnp.float32),
            pltpu.VMEM((NH, D), jnp.float32),
        ],
    )()


def _add_pos(rows_v, pos_v):
    def add_row(r, _):
        def add_slice(j, _):
            off = j * 16
            rows_v[r, pl.ds(off, 16)] = (
                rows_v[r, pl.ds(off, 16)] + pos_v[r, pl.ds(off, 16)]
            )
            return 0

        lax.fori_loop(0, D // 16, add_slice, 0, unroll=4)
        return 0

    lax.fori_loop(0, C, add_row, 0)


def _sc_body(
    pos_hbm, x_hbm, tab_hbm, out_hbm, idx_v, pos_v, r0, r1, g0, g1, sem
):
    cid = lax.axis_index("c")
    sid = lax.axis_index("s")
    wid = sid * NC + cid
    w0 = wid * S_PER_W

    rows = (r0, r1)
    gsem = (g0, g1)

    def gather(ch, b, buf):
        idx = idx_v.at[b, pl.ds(pl.multiple_of(ch * C, C), C)]
        pltpu.make_async_copy(tab_hbm.at[idx], rows[buf], gsem[buf]).start()

    def gather_wait(b, buf):
        idx = idx_v.at[b, pl.ds(0, C)]
        pltpu.make_async_copy(tab_hbm.at[idx], rows[buf], gsem[buf]).wait()

    for b in range(B):
        pltpu.sync_copy(
            x_hbm.at[pl.ds(pl.multiple_of(b * S + w0, C), S_PER_W)],
            idx_v.at[b],
        )
    pltpu.sync_copy(pos_hbm.at[pl.ds(pl.multiple_of(w0, C), C), :], pos_v)
    gather(0, 0, 0)

    def chunk_body(i, _):
        s0 = pl.multiple_of(w0 + i * C, C)
        for k in range(B):
            buf = k % 2
            nbuf = 1 - buf
            if k < B - 1:
                gather(i, k + 1, nbuf)
            else:

                @pl.when(i < NCH - 1)
                def _():
                    gather(i + 1, 0, nbuf)

            if k == 0:

                @pl.when(i > 0)
                def _():
                    pltpu.sync_copy(pos_hbm.at[pl.ds(s0, C), :], pos_v)

            gather_wait(k, buf)
            _add_pos(rows[buf], pos_v)
            base = pl.multiple_of(k * S + s0, C)
            pltpu.sync_copy(rows[buf], out_hbm.at[pl.ds(base, C), :])
        return 0

    lax.fori_loop(0, NCH, chunk_body, 0)


def _make_sc():
    mesh = plsc.VectorSubcoreMesh(core_axis_name="c", subcore_axis_name="s")
    return functools.partial(
        pl.kernel,
        mesh=mesh,
        out_type=jax.ShapeDtypeStruct((B * S, D), jnp.float32),
        scratch_types=[
            pltpu.VMEM((B, S_PER_W), jnp.int32),
            pltpu.VMEM((C, D), jnp.float32),
            pltpu.VMEM((C, D), jnp.float32),
            pltpu.VMEM((C, D), jnp.float32),
            pltpu.SemaphoreType.DMA,
            pltpu.SemaphoreType.DMA,
            pltpu.SemaphoreType.DMA,
        ],
    )(_sc_body)


@jax.jit
def kernel(x, tok_table):
    pos = _make_pos()
    out = _make_sc()(pos, x.reshape(B * S), tok_table)
    return out.reshape(B, S, D)


# speedup vs baseline: 4.0055x; 1.0106x over previous
import functools
import math

import jax
import jax.numpy as jnp
from jax import lax
from jax.experimental import pallas as pl
from jax.experimental.pallas import tpu as pltpu
from jax.experimental.pallas import tpu_sc as plsc

VOCAB = 100000
D = 1024
S = 8192
B = 4

NC = 2
NS = 16
NW = NC * NS
S_PER_W = S // NW
C = 32
NCH = S_PER_W // C

NH = S // 64
NL = 64

_LN10000 = math.log(10000.0)


_HPB = 16
_PBS = _HPB * NL


def _inv_denom_cols(shape, dim):
    col = lax.broadcasted_iota(jnp.int32, shape, dim)
    i_val = ((col >> 1) << 1).astype(jnp.float32)
    return jnp.exp(i_val * (_LN10000 / float(D)))


def _pos_body(out_ref, sinl_ref, cosl_ref, sinh_ref, cosh_ref):
    g = pl.program_id(0)

    @pl.when(g == 0)
    def _():
        al = lax.broadcasted_iota(jnp.int32, (NL, D), 0).astype(
            jnp.float32
        ) / _inv_denom_cols((NL, D), 1)
        sinl_ref[...] = jnp.sin(al)
        cosl_ref[...] = jnp.cos(al)
        ah = (
            lax.broadcasted_iota(jnp.int32, (NH, D), 0) * 64
        ).astype(jnp.float32) / _inv_denom_cols((NH, D), 1)
        sinh_ref[...] = jnp.sin(ah)
        cosh_ref[...] = jnp.cos(ah)

    hs = pl.ds(g * _HPB, _HPB)
    sh = sinh_ref[hs, :][:, None, :]
    ch_ = cosh_ref[hs, :][:, None, :]
    sl = sinl_ref[...][None, :, :]
    cl = cosl_ref[...][None, :, :]
    col = lax.broadcasted_iota(jnp.int32, (_HPB, NL, D), 2)
    even = (col & 1) == 0
    res = jnp.where(even, sh * cl + ch_ * sl, ch_ * cl - sh * sl)
    out_ref[...] = res.reshape(_PBS, D)


def _make_pos():
    return pl.pallas_call(
        _pos_body,
        out_shape=jax.ShapeDtypeStruct((S, D), jnp.float32),
        grid=(S // _PBS,),
        out_specs=pl.BlockSpec((_PBS, D), lambda g: (g, 0)),
        scratch_shapes=[
            pltpu.VMEM((NL, D), jnp.float32),
            pltpu.VMEM((NL, D), jnp.float32),
            pltpu.VMEM((NH, D), jnp.float32),
            pltpu.VMEM((NH, D), jnp.float32),
        ],
    )()


def _add_pos(rows_v, pos_v):
    def add_row(r, _):
        def add_slice(j, _):
            off = j * 16
            rows_v[r, pl.ds(off, 16)] = (
                rows_v[r, pl.ds(off, 16)] + pos_v[r, pl.ds(off, 16)]
            )
            return 0

        lax.fori_loop(0, D // 16, add_slice, 0, unroll=4)
        return 0

    lax.fori_loop(0, C, add_row, 0)


def _sc_body(
    pos_hbm, x_hbm, tab_hbm, out_hbm, idx_v, pos_v, r0, r1, g0, g1, sem
):
    cid = lax.axis_index("c")
    sid = lax.axis_index("s")
    wid = sid * NC + cid
    w0 = wid * S_PER_W

    rows = (r0, r1)
    gsem = (g0, g1)

    def gather(ch, b, buf):
        idx = idx_v.at[b, pl.ds(pl.multiple_of(ch * C, C), C)]
        pltpu.make_async_copy(tab_hbm.at[idx], rows[buf], gsem[buf]).start()

    def gather_wait(b, buf):
        idx = idx_v.at[b, pl.ds(0, C)]
        pltpu.make_async_copy(tab_hbm.at[idx], rows[buf], gsem[buf]).wait()

    for b in range(B):
        pltpu.sync_copy(
            x_hbm.at[pl.ds(pl.multiple_of(b * S + w0, C), S_PER_W)],
            idx_v.at[b],
        )
    pltpu.sync_copy(pos_hbm.at[pl.ds(pl.multiple_of(w0, C), C), :], pos_v)
    gather(0, 0, 0)

    def chunk_body(i, _):
        s0 = pl.multiple_of(w0 + i * C, C)
        for k in range(B):
            buf = k % 2
            nbuf = 1 - buf
            if k < B - 1:
                gather(i, k + 1, nbuf)
            else:

                @pl.when(i < NCH - 1)
                def _():
                    gather(i + 1, 0, nbuf)

            if k == 0:

                @pl.when(i > 0)
                def _():
                    pltpu.sync_copy(pos_hbm.at[pl.ds(s0, C), :], pos_v)

            gather_wait(k, buf)
            _add_pos(rows[buf], pos_v)
            base = pl.multiple_of(k * S + s0, C)
            pltpu.sync_copy(rows[buf], out_hbm.at[pl.ds(base, C), :])
        return 0

    lax.fori_loop(0, NCH, chunk_body, 0)


def _make_sc():
    mesh = plsc.VectorSubcoreMesh(core_axis_name="c", subcore_axis_name="s")
    return functools.partial(
        pl.kernel,
        mesh=mesh,
        out_type=jax.ShapeDtypeStruct((B * S, D), jnp.float32),
        scratch_types=[
            pltpu.VMEM((B, S_PER_W), jnp.int32),
            pltpu.VMEM((C, D), jnp.float32),
            pltpu.VMEM((C, D), jnp.float32),
            pltpu.VMEM((C, D), jnp.float32),
            pltpu.SemaphoreType.DMA,
            pltpu.SemaphoreType.DMA,
            pltpu.SemaphoreType.DMA,
        ],
    )(_sc_body)


@jax.jit
def kernel(x, tok_table):
    pos = _make_pos()
    out = _make_sc()(pos, x.reshape(B * S), tok_table)
    return out.reshape(B, S, D)


# speedup vs baseline: 4.0419x; 1.0091x over previous
import functools
import math

import jax
import jax.numpy as jnp
from jax import lax
from jax.experimental import pallas as pl
from jax.experimental.pallas import tpu as pltpu
from jax.experimental.pallas import tpu_sc as plsc

VOCAB = 100000
D = 1024
S = 8192
B = 4

NC = 2
NS = 16
NW = NC * NS
S_PER_W = S // NW
C = 32
NCH = S_PER_W // C

NH = S // 64
NL = 64

_LN10000 = math.log(10000.0)


_HPB = 16
_PBS = _HPB * NL


def _inv_denom_cols(shape, dim):
    col = lax.broadcasted_iota(jnp.int32, shape, dim)
    i_val = ((col >> 1) << 1).astype(jnp.float32)
    return jnp.exp(i_val * (_LN10000 / float(D)))


def _pos_body(out_ref, sinl_ref, cosl_ref, a_ref, b_ref):
    g = pl.program_id(0)

    @pl.when(g == 0)
    def _():
        al = lax.broadcasted_iota(jnp.int32, (NL, D), 0).astype(
            jnp.float32
        ) / _inv_denom_cols((NL, D), 1)
        sinl_ref[...] = jnp.sin(al)
        cosl_ref[...] = jnp.cos(al)
        ah = (
            lax.broadcasted_iota(jnp.int32, (NH, D), 0) * 64
        ).astype(jnp.float32) / _inv_denom_cols((NH, D), 1)
        sh = jnp.sin(ah)
        ch_ = jnp.cos(ah)
        col = lax.broadcasted_iota(jnp.int32, (NH, D), 1)
        even = (col & 1) == 0
        a_ref[...] = jnp.where(even, sh, ch_)
        b_ref[...] = jnp.where(even, ch_, -sh)

    hs = pl.ds(g * _HPB, _HPB)
    a = a_ref[hs, :][:, None, :]
    b = b_ref[hs, :][:, None, :]
    sl = sinl_ref[...][None, :, :]
    cl = cosl_ref[...][None, :, :]
    res = a * cl + b * sl
    out_ref[...] = res.reshape(_PBS, D)


def _make_pos():
    return pl.pallas_call(
        _pos_body,
        out_shape=jax.ShapeDtypeStruct((S, D), jnp.float32),
        grid=(S // _PBS,),
        out_specs=pl.BlockSpec((_PBS, D), lambda g: (g, 0)),
        scratch_shapes=[
            pltpu.VMEM((NL, D), jnp.float32),
            pltpu.VMEM((NL, D), jnp.float32),
            pltpu.VMEM((NH, D), jnp.float32),
            pltpu.VMEM((NH, D), jnp.float32),
        ],
    )()


def _add_pos(rows_v, pos_v):
    def add_row(r, _):
        def add_slice(j, _):
            off = j * 16
            rows_v[r, pl.ds(off, 16)] = (
                rows_v[r, pl.ds(off, 16)] + pos_v[r, pl.ds(off, 16)]
            )
            return 0

        lax.fori_loop(0, D // 16, add_slice, 0, unroll=4)
        return 0

    lax.fori_loop(0, C, add_row, 0)


def _sc_body(
    pos_hbm, x_hbm, tab_hbm, out_hbm, idx_v, pos_v, r0, r1, g0, g1, sem
):
    cid = lax.axis_index("c")
    sid = lax.axis_index("s")
    wid = sid * NC + cid
    w0 = wid * S_PER_W

    rows = (r0, r1)
    gsem = (g0, g1)

    def gather(ch, b, buf):
        idx = idx_v.at[b, pl.ds(pl.multiple_of(ch * C, C), C)]
        pltpu.make_async_copy(tab_hbm.at[idx], rows[buf], gsem[buf]).start()

    def gather_wait(b, buf):
        idx = idx_v.at[b, pl.ds(0, C)]
        pltpu.make_async_copy(tab_hbm.at[idx], rows[buf], gsem[buf]).wait()

    for b in range(B):
        pltpu.sync_copy(
            x_hbm.at[pl.ds(pl.multiple_of(b * S + w0, C), S_PER_W)],
            idx_v.at[b],
        )
    pltpu.sync_copy(pos_hbm.at[pl.ds(pl.multiple_of(w0, C), C), :], pos_v)
    gather(0, 0, 0)

    def chunk_body(i, _):
        s0 = pl.multiple_of(w0 + i * C, C)
        for k in range(B):
            buf = k % 2
            nbuf = 1 - buf
            if k < B - 1:
                gather(i, k + 1, nbuf)
            else:

                @pl.when(i < NCH - 1)
                def _():
                    gather(i + 1, 0, nbuf)

            if k == 0:

                @pl.when(i > 0)
                def _():
                    pltpu.sync_copy(pos_hbm.at[pl.ds(s0, C), :], pos_v)

            gather_wait(k, buf)
            _add_pos(rows[buf], pos_v)
            base = pl.multiple_of(k * S + s0, C)
            pltpu.sync_copy(rows[buf], out_hbm.at[pl.ds(base, C), :])
        return 0

    lax.fori_loop(0, NCH, chunk_body, 0)


def _make_sc():
    mesh = plsc.VectorSubcoreMesh(core_axis_name="c", subcore_axis_name="s")
    return functools.partial(
        pl.kernel,
        mesh=mesh,
        out_type=jax.ShapeDtypeStruct((B * S, D), jnp.float32),
        scratch_types=[
            pltpu.VMEM((B, S_PER_W), jnp.int32),
            pltpu.VMEM((C, D), jnp.float32),
            pltpu.VMEM((C, D), jnp.float32),
            pltpu.VMEM((C, D), jnp.float32),
            pltpu.SemaphoreType.DMA,
            pltpu.SemaphoreType.DMA,
            pltpu.SemaphoreType.DMA,
        ],
    )(_sc_body)


@jax.jit
def kernel(x, tok_table):
    pos = _make_pos()
    out = _make_sc()(pos, x.reshape(B * S), tok_table)
    return out.reshape(B, S, D)


# speedup vs baseline: 5.0964x; 1.2609x over previous
import functools
import math

import jax
import jax.numpy as jnp
from jax import lax
from jax.experimental import pallas as pl
from jax.experimental.pallas import tpu as pltpu
from jax.experimental.pallas import tpu_sc as plsc

VOCAB = 100000
D = 1024
S = 8192
B = 4

NC = 2
NS = 16
NW = NC * NS
S_PER_W = S // NW
C = 16
NCH = S_PER_W // C
NT = NCH * B

NH = S // 64
NL = 64

_LN10000 = math.log(10000.0)


_HPB = 16
_PBS = _HPB * NL


def _inv_denom_cols(shape, dim):
    col = lax.broadcasted_iota(jnp.int32, shape, dim)
    i_val = ((col >> 1) << 1).astype(jnp.float32)
    return jnp.exp(i_val * (_LN10000 / float(D)))


def _pos_body(out_ref, sinl_ref, cosl_ref, a_ref, b_ref):
    g = pl.program_id(0)

    @pl.when(g == 0)
    def _():
        al = lax.broadcasted_iota(jnp.int32, (NL, D), 0).astype(
            jnp.float32
        ) / _inv_denom_cols((NL, D), 1)
        sinl_ref[...] = jnp.sin(al)
        cosl_ref[...] = jnp.cos(al)
        ah = (
            lax.broadcasted_iota(jnp.int32, (NH, D), 0) * 64
        ).astype(jnp.float32) / _inv_denom_cols((NH, D), 1)
        sh = jnp.sin(ah)
        ch_ = jnp.cos(ah)
        col = lax.broadcasted_iota(jnp.int32, (NH, D), 1)
        even = (col & 1) == 0
        a_ref[...] = jnp.where(even, sh, ch_)
        b_ref[...] = jnp.where(even, ch_, -sh)

    hs = pl.ds(g * _HPB, _HPB)
    a = a_ref[hs, :][:, None, :]
    b = b_ref[hs, :][:, None, :]
    sl = sinl_ref[...][None, :, :]
    cl = cosl_ref[...][None, :, :]
    res = a * cl + b * sl
    out_ref[...] = res.reshape(_PBS, D)


def _make_pos():
    return pl.pallas_call(
        _pos_body,
        out_shape=jax.ShapeDtypeStruct((S, D), jnp.float32),
        grid=(S // _PBS,),
        out_specs=pl.BlockSpec((_PBS, D), lambda g: (g, 0)),
        scratch_shapes=[
            pltpu.VMEM((NL, D), jnp.float32),
            pltpu.VMEM((NL, D), jnp.float32),
            pltpu.VMEM((NH, D), jnp.float32),
            pltpu.VMEM((NH, D), jnp.float32),
        ],
    )()


def _add_pos(rows_v, pos_v):
    def add_row(r, _):
        def add_slice(j, _):
            off = j * 16
            rows_v[r, pl.ds(off, 16)] = (
                rows_v[r, pl.ds(off, 16)] + pos_v[r, pl.ds(off, 16)]
            )
            return 0

        lax.fori_loop(0, D // 16, add_slice, 0, unroll=4)
        return 0

    lax.fori_loop(0, C, add_row, 0)


def _sc_body(
    pos_hbm, x_hbm, tab_hbm, out_hbm,
    idx_v, p0, p1, r0, r1, r2, r3,
    g0, g1, g2, g3, s0_, s1_, s2_, s3_, ps0, ps1,
):
    cid = lax.axis_index("c")
    sid = lax.axis_index("s")
    wid = sid * NC + cid
    w0 = wid * S_PER_W

    rows = (r0, r1, r2, r3)
    gsem = (g0, g1, g2, g3)
    ssem = (s0_, s1_, s2_, s3_)
    pos = (p0, p1)
    psem = (ps0, ps1)

    def gather_start(ch, b, buf):
        idx = idx_v.at[b, pl.ds(pl.multiple_of(ch * C, C), C)]
        pltpu.make_async_copy(tab_hbm.at[idx], rows[buf], gsem[buf]).start()

    def gather_wait(buf):
        idx = idx_v.at[0, pl.ds(0, C)]
        pltpu.make_async_copy(tab_hbm.at[idx], rows[buf], gsem[buf]).wait()

    def store_start(ch, b, buf):
        base = pl.multiple_of(b * S + w0 + ch * C, C)
        pltpu.make_async_copy(
            rows[buf], out_hbm.at[pl.ds(base, C), :], ssem[buf]
        ).start()

    def store_wait(buf):
        pltpu.make_async_copy(
            rows[buf], out_hbm.at[pl.ds(w0, C), :], ssem[buf]
        ).wait()

    def pos_start(ch, pbuf):
        src = pos_hbm.at[pl.ds(pl.multiple_of(w0 + ch * C, C), C), :]
        pltpu.make_async_copy(src, pos[pbuf], psem[pbuf]).start()

    def pos_wait(pbuf):
        src = pos_hbm.at[pl.ds(pl.multiple_of(w0, C), C), :]
        pltpu.make_async_copy(src, pos[pbuf], psem[pbuf]).wait()

    for b in range(B):
        pltpu.sync_copy(
            x_hbm.at[pl.ds(pl.multiple_of(b * S + w0, C), S_PER_W)],
            idx_v.at[b],
        )
    pos_start(0, 0)
    pos_start(1, 1)
    gather_start(0, 0, 0)

    def body(i, _):
        for half in range(2):
            cc = i * 2 + half
            pbuf = half
            for k in range(B):
                if half == 0 and k < 3:

                    @pl.when(i > 0)
                    def _():
                        store_wait((k + 1) % 4)

                else:
                    store_wait((k + 1) % 4)
                if k < B - 1:
                    gather_start(cc, k + 1, k + 1)
                else:

                    @pl.when(cc < NCH - 1)
                    def _():
                        gather_start(cc + 1, 0, 0)

                if k == 0:
                    pos_wait(pbuf)
                gather_wait(k)
                _add_pos(rows[k], pos[pbuf])
                store_start(cc, k, k)
                if k == B - 1:

                    @pl.when(cc + 2 < NCH)
                    def _():
                        pos_start(cc + 2, pbuf)

        return 0

    lax.fori_loop(0, NCH // 2, body, 0)
    store_wait(1)
    store_wait(2)
    store_wait(3)


def _make_sc():
    mesh = plsc.VectorSubcoreMesh(core_axis_name="c", subcore_axis_name="s")
    f32 = jnp.float32
    return functools.partial(
        pl.kernel,
        mesh=mesh,
        out_type=jax.ShapeDtypeStruct((B * S, D), f32),
        scratch_types=[
            pltpu.VMEM((B, S_PER_W), jnp.int32),
            pltpu.VMEM((C, D), f32),
            pltpu.VMEM((C, D), f32),
            pltpu.VMEM((C, D), f32),
            pltpu.VMEM((C, D), f32),
            pltpu.VMEM((C, D), f32),
            pltpu.VMEM((C, D), f32),
        ]
        + [pltpu.SemaphoreType.DMA] * 10,
    )(_sc_body)


@jax.jit
def kernel(x, tok_table):
    pos = _make_pos()
    out = _make_sc()(pos, x.reshape(B * S), tok_table)
    return out.reshape(B, S, D)
